# baseline (speedup 1.0000x reference)
import functools

import jax
import jax.numpy as jnp
from jax import lax
from jax.experimental import pallas as pl
from jax.experimental.pallas import tpu as pltpu
from jax.experimental.pallas import tpu_sc as plsc

_MAXEMBED = 8192
_CIO = 1024
_HALF = _CIO // 2
_BATCH = 4
_SEQ = 4096
_MAX_NORM = 2.0

_L = 16
_NCORES = 2
_NSUB = 16
_NW = _NCORES * _NSUB
_GPB = _NW // _BATCH
_ROWS_PER_W = _SEQ // _GPB
_R = 32
_NCHUNK = _ROWS_PER_W // _R


def _rsqrt_nr(x):
    xi = lax.bitcast_convert_type(x, jnp.int32)
    yi = jnp.int32(0x5F3759DF) - lax.shift_right_logical(xi, 1)
    y = lax.bitcast_convert_type(yi, jnp.float32)
    hx = x * 0.5
    for _ in range(3):
        y = y * (1.5 - hx * y * y)
    return y


def _sc_body(pos_hbm, w0_hbm, w1_hbm, out_hbm, pos_v, a_v, b_v, o_v,
             sem_a, sem_b):
    c = lax.axis_index("c")
    s = lax.axis_index("s")
    wid = c * _NSUB + s
    b = wid // _GPB
    g = wid % _GPB
    row0 = g * _ROWS_PER_W

    pltpu.sync_copy(pos_hbm, pos_v)
    lanes_b = jnp.full((_L,), b, jnp.int32)
    p0v = plsc.load_gather(pos_v, [lanes_b, jnp.zeros((_L,), jnp.int32)])
    p1v = plsc.load_gather(pos_v, [lanes_b, jnp.ones((_L,), jnp.int32)])
    p0 = jnp.minimum(jnp.maximum(jnp.max(p0v), 0), _MAXEMBED + 1 - _SEQ)
    p1 = jnp.minimum(jnp.maximum(jnp.max(p1v), 0), _MAXEMBED + 1 - _SEQ)

    nvec = _HALF // _L

    def do_half(src_ref, r, col0):
        vals = [src_ref[r, pl.ds(j * _L, _L)] for j in range(nvec)]
        accs = [vals[k] * vals[k] for k in range(4)]
        for j in range(4, nvec):
            accs[j % 4] = accs[j % 4] + vals[j] * vals[j]
        acc = (accs[0] + accs[1]) + (accs[2] + accs[3])
        ssq = jnp.sum(acc)
        ssqv = jnp.full((_L,), ssq, jnp.float32)
        y = _rsqrt_nr(ssqv)
        scale = jnp.where(ssqv > _MAX_NORM * _MAX_NORM, _MAX_NORM * y,
                          jnp.float32(1.0))
        for j in range(nvec):
            o_v[r, pl.ds(col0 + j * _L, _L)] = vals[j] * scale

    def row_body(r, carry):
        do_half(a_v, r, 0)
        do_half(b_v, r, _HALF)
        return carry

    def chunk_body(i, carry):
        r0 = row0 + i * _R
        cp_a = pltpu.make_async_copy(w0_hbm.at[pl.ds(p0 + r0, _R)], a_v, sem_a)
        cp_b = pltpu.make_async_copy(w1_hbm.at[pl.ds(p1 + r0, _R)], b_v, sem_b)
        cp_a.start()
        cp_b.start()
        cp_a.wait()
        cp_b.wait()
        lax.fori_loop(0, _R, row_body, 0)
        pltpu.sync_copy(o_v, out_hbm.at[b, pl.ds(r0, _R)])
        return carry

    lax.fori_loop(0, _NCHUNK, chunk_body, 0)


_pos_embed = functools.partial(
    pl.kernel,
    out_type=jax.ShapeDtypeStruct((_BATCH, _SEQ, _CIO), jnp.float32),
    mesh=plsc.VectorSubcoreMesh(core_axis_name="c", subcore_axis_name="s"),
    scratch_types=[
        pltpu.VMEM((_BATCH, 2), jnp.int32),
        pltpu.VMEM((_R, _HALF), jnp.float32),
        pltpu.VMEM((_R, _HALF), jnp.float32),
        pltpu.VMEM((_R, _CIO), jnp.float32),
        pltpu.SemaphoreType.DMA,
        pltpu.SemaphoreType.DMA,
    ],
    compiler_params=pltpu.CompilerParams(
        use_tc_tiling_on_sc=False, needs_layout_passes=False),
)(_sc_body)


@jax.jit
def kernel(x, pos, W0, W1):
    del x
    return _pos_embed(pos, W0, W1)
